# baseline (speedup 1.0000x reference)
import functools

import jax
import jax.numpy as jnp
from jax import lax
from jax.experimental import pallas as pl
from jax.experimental.pallas import tpu as pltpu
from jax.experimental.pallas import tpu_sc as plsc

_NUM_TOKENS = 8192
_D = 32
_B = 64
_T = 1024
_N = _B * _T
_NC = 2
_NS = 16
_NW = _NC * _NS
_PER_W = _N // _NW
_ROWS_W = _B // _NW
_CHUNK = 128
_NCHUNK = _PER_W // _CHUNK
_CPR = _T // _CHUNK
_L = 16

_mesh = plsc.VectorSubcoreMesh(core_axis_name="c", subcore_axis_name="s")


@functools.partial(
    pl.kernel,
    mesh=_mesh,
    out_type=jax.ShapeDtypeStruct((_B, _T, _D), jnp.float32),
    scratch_types=[
        pltpu.VMEM((_NCHUNK, _CHUNK), jnp.int32),
        pltpu.VMEM((2, _CHUNK, 128), jnp.float32),
        pltpu.VMEM((2, _CHUNK, _D), jnp.float32),
        pltpu.SemaphoreType.DMA,
        pltpu.SemaphoreType.DMA,
    ],
)
def _gather_kernel(idx_hbm, table_hbm, out_hbm, idx_v, bufs, packs, gsem, ssem):
    wid = lax.axis_index("s") * _NC + lax.axis_index("c")
    pltpu.sync_copy(idx_hbm.at[pl.ds(wid * _NCHUNK, _NCHUNK)], idx_v)

    def fire_gather(j):
        return pltpu.async_copy(
            table_hbm.at[idx_v.at[j]], bufs.at[j % 2], gsem
        )

    def compact(b):
        def body(r, _):
            for h in range(_D // _L):
                packs[b, r, pl.ds(h * _L, _L)] = bufs[b, r, pl.ds(h * _L, _L)]
            return _

        lax.fori_loop(0, _CHUNK, body, None, unroll=8)

    def fire_store(j):
        bb = wid * _ROWS_W + j // _CPR
        t0 = (j % _CPR) * _CHUNK
        return pltpu.async_copy(
            packs.at[j % 2], out_hbm.at[bb, pl.ds(t0, _CHUNK)], ssem
        )

    gathers = [None] * _NCHUNK
    stores = [None] * _NCHUNK
    gathers[0] = fire_gather(0)
    gathers[1] = fire_gather(1)
    for j in range(_NCHUNK):
        b = j % 2
        gathers[j].wait()
        if j >= 2:
            stores[j - 2].wait()
        compact(b)
        if j + 2 < _NCHUNK:
            gathers[j + 2] = fire_gather(j + 2)
        stores[j] = fire_store(j)
    stores[_NCHUNK - 2].wait()
    stores[_NCHUNK - 1].wait()


def kernel(embed_id, weight):
    idx2 = embed_id.reshape(_N // _CHUNK, _CHUNK)
    table128 = jnp.pad(weight, ((0, 0), (0, 128 - _D)))
    return _gather_kernel(idx2, table128)


# speedup vs baseline: 1.0649x; 1.0649x over previous
import functools

import jax
import jax.numpy as jnp
from jax import lax
from jax.experimental import pallas as pl
from jax.experimental.pallas import tpu as pltpu
from jax.experimental.pallas import tpu_sc as plsc

_NUM_TOKENS = 8192
_D = 32
_B = 64
_T = 1024
_N = _B * _T
_NC = 2
_NS = 16
_NW = _NC * _NS
_PER_W = _N // _NW
_CHUNK = 128
_NCHUNK = _PER_W // _CHUNK
_RPC = _CHUNK * _D // 128
_PROWS_W = _PER_W * _D // 128
_NBUF = 4
_L = 16

_mesh = plsc.VectorSubcoreMesh(core_axis_name="c", subcore_axis_name="s")


@functools.partial(
    pl.kernel,
    mesh=_mesh,
    out_type=jax.ShapeDtypeStruct((_N * _D // 128, 128), jnp.float32),
    scratch_types=[
        pltpu.VMEM((_NCHUNK, _CHUNK), jnp.int32),
        pltpu.VMEM((_NBUF, _CHUNK, _D), jnp.float32),
        pltpu.VMEM((_PROWS_W, 128), jnp.float32),
        pltpu.SemaphoreType.DMA,
    ],
    compiler_params=pltpu.CompilerParams(use_tc_tiling_on_sc=False),
)
def _gather_kernel(idx_hbm, table_hbm, out_hbm, idx_v, bufs, packs, gsem):
    wid = lax.axis_index("s") * _NC + lax.axis_index("c")
    pltpu.sync_copy(idx_hbm.at[pl.ds(wid * _NCHUNK, _NCHUNK)], idx_v)

    def fire_gather(j):
        return pltpu.async_copy(
            table_hbm.at[idx_v.at[j]], bufs.at[j % _NBUF], gsem
        )

    def repack(j):
        b = j % _NBUF

        def body(q, _):
            for u in range(128 // _D):
                for h in range(_D // _L):
                    packs[j * _RPC + q, pl.ds(u * _D + h * _L, _L)] = (
                        bufs[b, q * (128 // _D) + u, pl.ds(h * _L, _L)]
                    )
            return _

        lax.fori_loop(0, _RPC, body, None, unroll=4)

    gathers = [None] * _NCHUNK
    for j in range(_NBUF):
        gathers[j] = fire_gather(j)
    for j in range(_NCHUNK):
        gathers[j].wait()
        repack(j)
        if j + _NBUF < _NCHUNK:
            gathers[j + _NBUF] = fire_gather(j + _NBUF)
    pltpu.sync_copy(packs, out_hbm.at[pl.ds(wid * _PROWS_W, _PROWS_W)])


def kernel(embed_id, weight):
    idx2 = embed_id.reshape(_N // _CHUNK, _CHUNK)
    out = _gather_kernel(idx2, weight)
    return out.reshape(_B, _T, _D)
